# baseline (speedup 1.0000x reference)
import jax
import jax.numpy as jnp
from jax import lax
from jax.experimental import pallas as pl
from jax.experimental.pallas import tpu as pltpu
from jax.experimental.pallas import tpu_sc as plsc

DIM = 64
NC = 2
NS = 16
NW = NC * NS
LANES = 16

CB = 8


def _body(tok_hbm, iw2_hbm, tv_hbm, w_hbm, b_hbm, out_hbm,
          idx_v, idx2_v, vals_v, pairs_v, rows_o, w_v, b_v, sem):
    bsz, fields = tok_hbm.shape
    wid = lax.axis_index("s") * NC + lax.axis_index("c")
    b_per_w = bsz // NW
    nchunks = b_per_w // CB
    b_base = wid * b_per_w
    ngrp = DIM // LANES

    pltpu.sync_copy(w_hbm, w_v)
    pltpu.sync_copy(b_hbm, b_v)
    wv = [w_v[pl.ds(g * LANES, LANES)] for g in range(ngrp)]
    bv = [b_v[pl.ds(g * LANES, LANES)] for g in range(ngrp)]

    lo_off = 0
    hi_off = fields - LANES

    def chunk_body(c, carry):
        b0 = b_base + c * CB
        pltpu.sync_copy(tok_hbm.at[pl.ds(b0, CB)], idx_v)
        for b in range(CB):
            lo = idx_v[b, pl.ds(lo_off, LANES)]
            hi = idx_v[b, pl.ds(hi_off, LANES)]
            idx2_v[b, pl.ds(lo_off, LANES)] = lax.shift_right_logical(lo, 1)
            idx2_v[b, pl.ds(hi_off, LANES)] = lax.shift_right_logical(hi, 1)
        cps = []
        for b in range(CB):
            cps.append(pltpu.async_copy(
                iw2_hbm.at[idx2_v.at[b]], pairs_v.at[b], sem))
            cps.append(pltpu.async_copy(
                tv_hbm.at[idx_v.at[b]], vals_v.at[b], sem))
        for cp in cps:
            cp.wait()

        def b_body(b, rcarry):
            tlo = idx_v[b, pl.ds(lo_off, LANES)]
            thi = idx_v[b, pl.ds(hi_off, LANES)]
            vlo = vals_v[b, pl.ds(lo_off, LANES)]
            vhi = vals_v[b, pl.ds(hi_off, LANES)]
            for f in range(fields):
                if f < LANES:
                    t = tlo[f]
                    val = vlo[f]
                else:
                    t = thi[f - hi_off]
                    val = vhi[f - hi_off]
                off = (t & 1) * DIM
                for g in range(ngrp):
                    rows_o[b, f, pl.ds(g * LANES, LANES)] = (
                        pairs_v[b, f, pl.ds(off + g * LANES, LANES)]
                        + (val * wv[g] + bv[g]))
            return rcarry
        lax.fori_loop(0, CB, b_body, 0)

        pltpu.sync_copy(rows_o, out_hbm.at[pl.ds(b0, CB)])
        return carry

    lax.fori_loop(0, nchunks, chunk_body, 0)


def kernel(tokens, index_weight, w1, b1, token_values):
    bsz, fields = tokens.shape
    nemb = index_weight.shape[0]
    iw2 = index_weight.reshape(nemb // 2, 2 * DIM)

    run = pl.kernel(
        _body,
        out_type=jax.ShapeDtypeStruct((bsz, fields, DIM), jnp.float32),
        mesh=plsc.VectorSubcoreMesh(core_axis_name="c", subcore_axis_name="s"),
        scratch_types=[
            pltpu.VMEM((CB, fields), jnp.int32),
            pltpu.VMEM((CB, fields), jnp.int32),
            pltpu.VMEM((CB, fields), jnp.float32),
            pltpu.VMEM((CB, fields, 2 * DIM), jnp.float32),
            pltpu.VMEM((CB, fields, DIM), jnp.float32),
            pltpu.VMEM((DIM,), jnp.float32),
            pltpu.VMEM((DIM,), jnp.float32),
            pltpu.SemaphoreType.DMA,
        ],
        compiler_params=pltpu.CompilerParams(use_tc_tiling_on_sc=True),
    )
    return run(tokens, iw2, token_values, w1[:, 0], b1)


# speedup vs baseline: 1.5214x; 1.5214x over previous
import jax
import jax.numpy as jnp
from jax import lax
from jax.experimental import pallas as pl
from jax.experimental.pallas import tpu as pltpu
from jax.experimental.pallas import tpu_sc as plsc

DIM = 64
NC = 2
NS = 16
NW = NC * NS
LANES = 16

CB = 32


def _body(tok_hbm, iw_hbm, tv_hbm, w_hbm, b_hbm, out_hbm,
          idx_v, vals_v, gbuf_v, rows_v, w_v, b_v, sem):
    bsz, fields = tok_hbm.shape
    wid = lax.axis_index("s") * NC + lax.axis_index("c")
    b_per_w = bsz // NW
    nchunks = b_per_w // CB
    b_base = wid * b_per_w
    ngrp = DIM // LANES
    fh = fields // 2

    pltpu.sync_copy(w_hbm, w_v)
    pltpu.sync_copy(b_hbm, b_v)
    wv = [w_v[pl.ds(g * LANES, LANES)] for g in range(ngrp)]
    bv = [b_v[pl.ds(g * LANES, LANES)] for g in range(ngrp)]

    lo_off = 0
    hi_off = fields - LANES

    def chunk_body(c, carry):
        b0 = b_base + c * CB
        pltpu.sync_copy(tok_hbm.at[pl.ds(b0, CB)], idx_v)
        cps = []
        for j in range(CB):
            cps.append(pltpu.async_copy(
                iw_hbm.at[idx_v.at[j]], gbuf_v.at[j], sem))
            cps.append(pltpu.async_copy(
                tv_hbm.at[idx_v.at[j]], vals_v.at[j], sem))
        for cp in cps:
            cp.wait()

        def b_body(b, rcarry):
            vlo = vals_v[b, pl.ds(lo_off, LANES)]
            vhi = vals_v[b, pl.ds(hi_off, LANES)]
            r2 = b * fh
            for f in range(fields):
                if f < LANES:
                    val = vlo[f]
                else:
                    val = vhi[f - hi_off]
                half = (f % 2) * DIM
                for g in range(ngrp):
                    rows_v[r2 + f // 2, pl.ds(half + g * LANES, LANES)] = (
                        gbuf_v[b, f, pl.ds(g * LANES, LANES)]
                        + (val * wv[g] + bv[g]))
            return rcarry
        lax.fori_loop(0, CB, b_body, 0)

        pltpu.sync_copy(rows_v, out_hbm.at[pl.ds(b0 * fh, CB * fh)])
        return carry

    lax.fori_loop(0, nchunks, chunk_body, 0)


def kernel(tokens, index_weight, w1, b1, token_values):
    bsz, fields = tokens.shape
    n2 = bsz * fields // 2

    run = pl.kernel(
        _body,
        out_type=jax.ShapeDtypeStruct((n2, 2 * DIM), jnp.float32),
        mesh=plsc.VectorSubcoreMesh(core_axis_name="c", subcore_axis_name="s"),
        scratch_types=[
            pltpu.VMEM((CB, fields), jnp.int32),
            pltpu.VMEM((CB, fields), jnp.float32),
            pltpu.VMEM((CB, fields, DIM), jnp.float32),
            pltpu.VMEM((CB * fields // 2, 2 * DIM), jnp.float32),
            pltpu.VMEM((DIM,), jnp.float32),
            pltpu.VMEM((DIM,), jnp.float32),
            pltpu.SemaphoreType.DMA,
        ],
        compiler_params=pltpu.CompilerParams(use_tc_tiling_on_sc=False),
    )
    out2 = run(tokens, index_weight, token_values, w1[:, 0], b1)
    return out2.reshape(bsz, fields, DIM)
